# baseline (speedup 1.0000x reference)
import functools

import jax
import jax.numpy as jnp
from jax import lax
from jax.experimental import pallas as pl
from jax.experimental.pallas import tpu as pltpu
from jax.experimental.pallas import tpu_sc as plsc

NC = 2
NS = 16
L = 16
NW = NC * NS
CH = 128

_mesh = lambda: plsc.VectorSubcoreMesh(core_axis_name="c", subcore_axis_name="s")


def _make_degrees(n_pad, nchunk):
    @functools.partial(
        pl.kernel,
        out_type=jax.ShapeDtypeStruct((2, NW, n_pad), jnp.float32),
        mesh=_mesh(),
        compiler_params=pltpu.CompilerParams(needs_layout_passes=False),
        scratch_types=[
            pltpu.VMEM((nchunk, CH), jnp.int32),
            pltpu.VMEM((n_pad,), jnp.float32),
            pltpu.VMEM((n_pad,), jnp.float32),
        ],
    )
    def deg_kernel(src_hbm, dst_hbm, out_hbm, idxv, hist_s, hist_d):
        cid = lax.axis_index("c")
        sid = lax.axis_index("s")
        wid = sid * NC + cid

        zeros16 = jnp.zeros((L,), jnp.float32)
        ones16 = jnp.ones((L,), jnp.float32)

        def zbody(i, carry):
            hist_s[pl.ds(i * L, L)] = zeros16
            hist_d[pl.ds(i * L, L)] = zeros16
            return carry

        lax.fori_loop(0, n_pad // L, zbody, 0)

        def accum(idx_hbm, hist):
            pltpu.sync_copy(idx_hbm.at[wid], idxv)

            def rbody(r, carry):
                def jbody(j, carry2):
                    iv = idxv[r, pl.ds(j * L, L)]
                    plsc.addupdate_scatter(hist, [iv], ones16)
                    return carry2
                return lax.fori_loop(0, CH // L, jbody, carry)

            lax.fori_loop(0, nchunk, rbody, 0)

        accum(src_hbm, hist_s)
        accum(dst_hbm, hist_d)
        pltpu.sync_copy(hist_s, out_hbm.at[0, wid])
        pltpu.sync_copy(hist_d, out_hbm.at[1, wid])

    return deg_kernel


def _make_scatter(n_pad, nchunk, d, slow_core=1):
    rows_per_tile = n_pad // NS
    tot = NW * nchunk
    t_slow = tot // (3 * NS)
    t_fast = (tot - NS * t_slow) // NS
    assert NS * (t_slow + t_fast) == tot
    fast_base = NS * t_slow

    @functools.partial(
        pl.kernel,
        out_type=jax.ShapeDtypeStruct((NC, n_pad, d), jnp.float32),
        mesh=_mesh(),
        compiler_params=pltpu.CompilerParams(
            needs_layout_passes=False, use_tc_tiling_on_sc=False),
        scratch_types=[
            pltpu.VMEM((t_fast, CH), jnp.int32),
            pltpu.VMEM((t_fast, CH), jnp.int32),
            pltpu.VMEM((CH, d), jnp.float32),
            pltpu.VMEM_SHARED((n_pad, d), jnp.float32),
            pltpu.SemaphoreType.DMA,
        ],
    )
    def scatter_kernel(z_hbm, src_hbm, dst_hbm, zero_hbm, out_hbm,
                       srcv, dstv, buf, agg, sem):
        cid = lax.axis_index("c")
        sid = lax.axis_index("s")
        r0 = sid * rows_per_tile
        is_slow = cid == slow_core
        base = jnp.where(is_slow, sid * t_slow, fast_base + sid * t_fast)
        cnt = jnp.where(is_slow, t_slow, t_fast)

        pltpu.sync_copy(zero_hbm.at[pl.ds(r0, rows_per_tile)],
                        agg.at[pl.ds(r0, rows_per_tile)])
        pltpu.sync_copy(src_hbm.at[pl.ds(base, t_fast)], srcv)
        pltpu.sync_copy(dst_hbm.at[pl.ds(base, t_fast)], dstv)
        plsc.subcore_barrier()

        def body(c, carry):
            pltpu.async_copy(z_hbm.at[srcv.at[c]], buf, sem).wait()
            pltpu.sync_copy(buf, agg.at[dstv.at[c]], add=True)
            return carry

        lax.fori_loop(0, cnt, body, 0)
        plsc.subcore_barrier()
        pltpu.sync_copy(agg.at[pl.ds(r0, rows_per_tile)],
                        out_hbm.at[cid, pl.ds(r0, rows_per_tile)])

    return scatter_kernel


def _stage_a(xp, degs, W1, n_pad, bm=256):
    d_in, h1 = W1.shape

    def body(x_ref, deg_ref, w_ref, z_ref, ns_ref, nd_ref):
        dsum = jnp.sum(deg_ref[...], axis=1)
        ns = lax.rsqrt(jnp.maximum(dsum[0], 1.0))
        nd = lax.rsqrt(jnp.maximum(dsum[1], 1.0))
        x = x_ref[...] * ns[:, None]
        z_ref[...] = jnp.dot(x, w_ref[...], preferred_element_type=jnp.float32)
        ns_ref[...] = ns
        nd_ref[...] = nd

    return pl.pallas_call(
        body,
        grid=(n_pad // bm,),
        in_specs=[
            pl.BlockSpec((bm, d_in), lambda i: (i, 0)),
            pl.BlockSpec((2, NW, bm), lambda i: (0, 0, i)),
            pl.BlockSpec((d_in, h1), lambda i: (0, 0)),
        ],
        out_specs=[
            pl.BlockSpec((bm, h1), lambda i: (i, 0)),
            pl.BlockSpec((bm,), lambda i: (i,)),
            pl.BlockSpec((bm,), lambda i: (i,)),
        ],
        out_shape=[
            jax.ShapeDtypeStruct((n_pad, h1), jnp.float32),
            jax.ShapeDtypeStruct((n_pad,), jnp.float32),
            jax.ShapeDtypeStruct((n_pad,), jnp.float32),
        ],
    )(xp, degs, W1)


def _stage_b(s1, ns, nd, b1, W2, n_pad, bm=256):
    h1, h2 = W2.shape

    def body(s_ref, ns_ref, nd_ref, b_ref, w_ref, z_ref):
        s = s_ref[0] + s_ref[1]
        hid = jnp.maximum(s * nd_ref[...][:, None] + b_ref[...][None, :], 0.0)
        hid = hid * ns_ref[...][:, None]
        z_ref[...] = jnp.dot(hid, w_ref[...], preferred_element_type=jnp.float32)

    return pl.pallas_call(
        body,
        grid=(n_pad // bm,),
        in_specs=[
            pl.BlockSpec((2, bm, h1), lambda i: (0, i, 0)),
            pl.BlockSpec((bm,), lambda i: (i,)),
            pl.BlockSpec((bm,), lambda i: (i,)),
            pl.BlockSpec((h1,), lambda i: (0,)),
            pl.BlockSpec((h1, h2), lambda i: (0, 0)),
        ],
        out_specs=pl.BlockSpec((bm, h2), lambda i: (i, 0)),
        out_shape=jax.ShapeDtypeStruct((n_pad, h2), jnp.float32),
    )(s1, ns, nd, b1, W2)


def _decoder(s2, nd, b2, n, bm=1024, bn=1024):
    h2 = s2.shape[-1]

    def body(sa_ref, sb_ref, na_ref, nb_ref, b_ref, o_ref):
        bias = b_ref[...][None, :]
        a = jnp.maximum((sa_ref[0] + sa_ref[1]) * na_ref[...][:, None] + bias, 0.0)
        b = jnp.maximum((sb_ref[0] + sb_ref[1]) * nb_ref[...][:, None] + bias, 0.0)
        logits = lax.dot_general(
            a, b, (((1,), (1,)), ((), ())),
            preferred_element_type=jnp.float32)
        o_ref[...] = jax.nn.sigmoid(logits)

    return pl.pallas_call(
        body,
        grid=(-(-n // bm), -(-n // bn)),
        in_specs=[
            pl.BlockSpec((2, bm, h2), lambda i, j: (0, i, 0)),
            pl.BlockSpec((2, bn, h2), lambda i, j: (0, j, 0)),
            pl.BlockSpec((bm,), lambda i, j: (i,)),
            pl.BlockSpec((bn,), lambda i, j: (j,)),
            pl.BlockSpec((h2,), lambda i, j: (0,)),
        ],
        out_specs=pl.BlockSpec((bm, bn), lambda i, j: (i, j)),
        out_shape=jax.ShapeDtypeStruct((n, n), jnp.float32),
    )(s2, s2, nd, nd, b2)


def kernel(features, edge_index, W1, b1, W2, b2):
    n, d_in = features.shape
    h1 = W1.shape[1]
    h2 = W2.shape[1]
    e = edge_index.shape[1]

    n_pad = -(-(n + 1) // 256) * 256
    nchunk = -(-e // (NW * CH))
    e_pad = NW * nchunk * CH

    src = edge_index[0].astype(jnp.int32)
    dst = edge_index[1].astype(jnp.int32)
    pad_idx = jnp.full((e_pad - e,), n, jnp.int32)
    srcp = jnp.concatenate([src, pad_idx]).reshape(NW, nchunk, CH)
    dstp = jnp.concatenate([dst, pad_idx]).reshape(NW, nchunk, CH)

    degs = _make_degrees(n_pad, nchunk)(srcp, dstp)

    z1, ns, nd = _stage_a(features, degs, W1, n_pad)

    src2d = srcp.reshape(NW * nchunk, CH)
    dst2d = dstp.reshape(NW * nchunk, CH)
    zeros1 = jnp.zeros((n_pad, h1), jnp.float32)
    s1 = _make_scatter(n_pad, nchunk, h1)(z1, src2d, dst2d, zeros1)

    z2 = _stage_b(s1, ns, nd, b1, W2, n_pad)

    zeros2 = jnp.zeros((n_pad, h2), jnp.float32)
    s2 = _make_scatter(n_pad, nchunk, h2)(z2, src2d, dst2d, zeros2)

    return _decoder(s2, nd, b2, n)


# speedup vs baseline: 1.0877x; 1.0877x over previous
import functools

import jax
import jax.numpy as jnp
from jax import lax
from jax.experimental import pallas as pl
from jax.experimental.pallas import tpu as pltpu
from jax.experimental.pallas import tpu_sc as plsc

NC = 2
NS = 16
L = 16
NW = NC * NS
CH = 128

_mesh = lambda: plsc.VectorSubcoreMesh(core_axis_name="c", subcore_axis_name="s")


def _make_degrees(n_pad, nchunk):
    @functools.partial(
        pl.kernel,
        out_type=jax.ShapeDtypeStruct((2, NW, n_pad), jnp.float32),
        mesh=_mesh(),
        compiler_params=pltpu.CompilerParams(needs_layout_passes=False),
        scratch_types=[
            pltpu.VMEM((nchunk, CH), jnp.int32),
            pltpu.VMEM((n_pad,), jnp.float32),
            pltpu.VMEM((n_pad,), jnp.float32),
        ],
    )
    def deg_kernel(src_hbm, dst_hbm, out_hbm, idxv, hist_s, hist_d):
        cid = lax.axis_index("c")
        sid = lax.axis_index("s")
        wid = sid * NC + cid

        zeros16 = jnp.zeros((L,), jnp.float32)
        ones16 = jnp.ones((L,), jnp.float32)

        def zbody(i, carry):
            hist_s[pl.ds(i * L, L)] = zeros16
            hist_d[pl.ds(i * L, L)] = zeros16
            return carry

        lax.fori_loop(0, n_pad // L, zbody, 0)

        def accum(idx_hbm, hist):
            pltpu.sync_copy(idx_hbm.at[wid], idxv)

            def rbody(r, carry):
                def jbody(j, carry2):
                    iv = idxv[r, pl.ds(j * L, L)]
                    plsc.addupdate_scatter(hist, [iv], ones16)
                    return carry2
                return lax.fori_loop(0, CH // L, jbody, carry)

            lax.fori_loop(0, nchunk, rbody, 0)

        accum(src_hbm, hist_s)
        accum(dst_hbm, hist_d)
        pltpu.sync_copy(hist_s, out_hbm.at[0, wid])
        pltpu.sync_copy(hist_d, out_hbm.at[1, wid])

    return deg_kernel


def _make_scatter(n_pad, nchunk, d, slow_core=1):
    rows_per_tile = n_pad // NS
    tot = NW * nchunk
    t_slow = tot // (2 * NS)
    t_fast = (tot - NS * t_slow) // NS
    assert NS * (t_slow + t_fast) == tot
    fast_base = NS * t_slow

    @functools.partial(
        pl.kernel,
        out_type=jax.ShapeDtypeStruct((NC, n_pad, d), jnp.float32),
        mesh=_mesh(),
        compiler_params=pltpu.CompilerParams(
            needs_layout_passes=False, use_tc_tiling_on_sc=False),
        scratch_types=[
            pltpu.VMEM((t_fast, CH), jnp.int32),
            pltpu.VMEM((t_fast, CH), jnp.int32),
            pltpu.VMEM((CH, d), jnp.float32),
            pltpu.VMEM_SHARED((n_pad, d), jnp.float32),
            pltpu.SemaphoreType.DMA,
        ],
    )
    def scatter_kernel(z_hbm, src_hbm, dst_hbm, zero_hbm, out_hbm,
                       srcv, dstv, buf, agg, sem):
        cid = lax.axis_index("c")
        sid = lax.axis_index("s")
        r0 = sid * rows_per_tile
        is_slow = cid == slow_core
        base = jnp.where(is_slow, sid * t_slow, fast_base + sid * t_fast)
        cnt = jnp.where(is_slow, t_slow, t_fast)

        pltpu.sync_copy(zero_hbm.at[pl.ds(r0, rows_per_tile)],
                        agg.at[pl.ds(r0, rows_per_tile)])
        pltpu.sync_copy(src_hbm.at[pl.ds(base, t_fast)], srcv)
        pltpu.sync_copy(dst_hbm.at[pl.ds(base, t_fast)], dstv)
        plsc.subcore_barrier()

        def body(c, carry):
            pltpu.async_copy(z_hbm.at[srcv.at[c]], buf, sem).wait()
            pltpu.sync_copy(buf, agg.at[dstv.at[c]], add=True)
            return carry

        lax.fori_loop(0, cnt, body, 0)
        plsc.subcore_barrier()
        pltpu.sync_copy(agg.at[pl.ds(r0, rows_per_tile)],
                        out_hbm.at[cid, pl.ds(r0, rows_per_tile)])

    return scatter_kernel


def _stage_a(xp, degs, W1, n_pad, bm=256):
    d_in, h1 = W1.shape

    def body(x_ref, deg_ref, w_ref, z_ref, ns_ref, nd_ref):
        dsum = jnp.sum(deg_ref[...], axis=1)
        ns = lax.rsqrt(jnp.maximum(dsum[0], 1.0))
        nd = lax.rsqrt(jnp.maximum(dsum[1], 1.0))
        x = x_ref[...] * ns[:, None]
        z_ref[...] = jnp.dot(x, w_ref[...], preferred_element_type=jnp.float32)
        ns_ref[...] = ns
        nd_ref[...] = nd

    return pl.pallas_call(
        body,
        grid=(n_pad // bm,),
        in_specs=[
            pl.BlockSpec((bm, d_in), lambda i: (i, 0)),
            pl.BlockSpec((2, NW, bm), lambda i: (0, 0, i)),
            pl.BlockSpec((d_in, h1), lambda i: (0, 0)),
        ],
        out_specs=[
            pl.BlockSpec((bm, h1), lambda i: (i, 0)),
            pl.BlockSpec((bm,), lambda i: (i,)),
            pl.BlockSpec((bm,), lambda i: (i,)),
        ],
        out_shape=[
            jax.ShapeDtypeStruct((n_pad, h1), jnp.float32),
            jax.ShapeDtypeStruct((n_pad,), jnp.float32),
            jax.ShapeDtypeStruct((n_pad,), jnp.float32),
        ],
    )(xp, degs, W1)


def _stage_b(s1, ns, nd, b1, W2, n_pad, bm=256):
    h1, h2 = W2.shape

    def body(s_ref, ns_ref, nd_ref, b_ref, w_ref, z_ref):
        s = s_ref[0] + s_ref[1]
        hid = jnp.maximum(s * nd_ref[...][:, None] + b_ref[...][None, :], 0.0)
        hid = hid * ns_ref[...][:, None]
        z_ref[...] = jnp.dot(hid, w_ref[...], preferred_element_type=jnp.float32)

    return pl.pallas_call(
        body,
        grid=(n_pad // bm,),
        in_specs=[
            pl.BlockSpec((2, bm, h1), lambda i: (0, i, 0)),
            pl.BlockSpec((bm,), lambda i: (i,)),
            pl.BlockSpec((bm,), lambda i: (i,)),
            pl.BlockSpec((h1,), lambda i: (0,)),
            pl.BlockSpec((h1, h2), lambda i: (0, 0)),
        ],
        out_specs=pl.BlockSpec((bm, h2), lambda i: (i, 0)),
        out_shape=jax.ShapeDtypeStruct((n_pad, h2), jnp.float32),
    )(s1, ns, nd, b1, W2)


def _decoder(s2, nd, b2, n, bm=1024, bn=1024):
    h2 = s2.shape[-1]

    def body(sa_ref, sb_ref, na_ref, nb_ref, b_ref, o_ref):
        bias = b_ref[...][None, :]
        a = jnp.maximum((sa_ref[0] + sa_ref[1]) * na_ref[...][:, None] + bias, 0.0)
        b = jnp.maximum((sb_ref[0] + sb_ref[1]) * nb_ref[...][:, None] + bias, 0.0)
        logits = lax.dot_general(
            a, b, (((1,), (1,)), ((), ())),
            preferred_element_type=jnp.float32)
        o_ref[...] = jax.nn.sigmoid(logits)

    return pl.pallas_call(
        body,
        grid=(-(-n // bm), -(-n // bn)),
        in_specs=[
            pl.BlockSpec((2, bm, h2), lambda i, j: (0, i, 0)),
            pl.BlockSpec((2, bn, h2), lambda i, j: (0, j, 0)),
            pl.BlockSpec((bm,), lambda i, j: (i,)),
            pl.BlockSpec((bn,), lambda i, j: (j,)),
            pl.BlockSpec((h2,), lambda i, j: (0,)),
        ],
        out_specs=pl.BlockSpec((bm, bn), lambda i, j: (i, j)),
        out_shape=jax.ShapeDtypeStruct((n, n), jnp.float32),
    )(s2, s2, nd, nd, b2)


def kernel(features, edge_index, W1, b1, W2, b2):
    n, d_in = features.shape
    h1 = W1.shape[1]
    h2 = W2.shape[1]
    e = edge_index.shape[1]

    n_pad = -(-(n + 1) // 256) * 256
    nchunk = -(-e // (NW * CH))
    e_pad = NW * nchunk * CH

    src = edge_index[0].astype(jnp.int32)
    dst = edge_index[1].astype(jnp.int32)
    pad_idx = jnp.full((e_pad - e,), n, jnp.int32)
    srcp = jnp.concatenate([src, pad_idx]).reshape(NW, nchunk, CH)
    dstp = jnp.concatenate([dst, pad_idx]).reshape(NW, nchunk, CH)

    degs = _make_degrees(n_pad, nchunk)(srcp, dstp)

    z1, ns, nd = _stage_a(features, degs, W1, n_pad)

    src2d = srcp.reshape(NW * nchunk, CH)
    dst2d = dstp.reshape(NW * nchunk, CH)
    zeros1 = jnp.zeros((n_pad, h1), jnp.float32)
    s1 = _make_scatter(n_pad, nchunk, h1)(z1, src2d, dst2d, zeros1)

    z2 = _stage_b(s1, ns, nd, b1, W2, n_pad)

    zeros2 = jnp.zeros((n_pad, h2), jnp.float32)
    s2 = _make_scatter(n_pad, nchunk, h2)(z2, src2d, dst2d, zeros2)

    return _decoder(s2, nd, b2, n)


# speedup vs baseline: 1.1030x; 1.0140x over previous
import functools

import jax
import jax.numpy as jnp
from jax import lax
from jax.experimental import pallas as pl
from jax.experimental.pallas import tpu as pltpu
from jax.experimental.pallas import tpu_sc as plsc

NC = 2
NS = 16
L = 16
NW = NC * NS
CH = 128

_mesh = lambda: plsc.VectorSubcoreMesh(core_axis_name="c", subcore_axis_name="s")


def _make_degrees(n_pad, nchunk):
    @functools.partial(
        pl.kernel,
        out_type=jax.ShapeDtypeStruct((2, NW, n_pad), jnp.float32),
        mesh=_mesh(),
        compiler_params=pltpu.CompilerParams(needs_layout_passes=False),
        scratch_types=[
            pltpu.VMEM((nchunk, CH), jnp.int32),
            pltpu.VMEM((n_pad,), jnp.float32),
            pltpu.VMEM((n_pad,), jnp.float32),
        ],
    )
    def deg_kernel(src_hbm, dst_hbm, out_hbm, idxv, hist_s, hist_d):
        cid = lax.axis_index("c")
        sid = lax.axis_index("s")
        wid = sid * NC + cid

        zeros16 = jnp.zeros((L,), jnp.float32)
        ones16 = jnp.ones((L,), jnp.float32)

        def zbody(i, carry):
            hist_s[pl.ds(i * L, L)] = zeros16
            hist_d[pl.ds(i * L, L)] = zeros16
            return carry

        lax.fori_loop(0, n_pad // L, zbody, 0)

        def accum(idx_hbm, hist):
            pltpu.sync_copy(idx_hbm.at[wid], idxv)

            def rbody(r, carry):
                def jbody(j, carry2):
                    iv = idxv[r, pl.ds(j * L, L)]
                    plsc.addupdate_scatter(hist, [iv], ones16)
                    return carry2
                return lax.fori_loop(0, CH // L, jbody, carry)

            lax.fori_loop(0, nchunk, rbody, 0)

        accum(src_hbm, hist_s)
        accum(dst_hbm, hist_d)
        pltpu.sync_copy(hist_s, out_hbm.at[0, wid])
        pltpu.sync_copy(hist_d, out_hbm.at[1, wid])

    return deg_kernel


def _make_scatter(n_pad, nchunk, d):
    rows_per_tile = n_pad // NS

    @functools.partial(
        pl.kernel,
        out_type=jax.ShapeDtypeStruct((NC, n_pad, d), jnp.float32),
        mesh=_mesh(),
        compiler_params=pltpu.CompilerParams(
            needs_layout_passes=False, use_tc_tiling_on_sc=False),
        scratch_types=[
            pltpu.VMEM((nchunk, CH), jnp.int32),
            pltpu.VMEM((nchunk, CH), jnp.int32),
            pltpu.VMEM((CH, d), jnp.float32),
            pltpu.VMEM_SHARED((n_pad, d), jnp.float32),
            pltpu.SemaphoreType.DMA,
        ],
    )
    def scatter_kernel(z_hbm, src_hbm, dst_hbm, zero_hbm, out_hbm,
                       srcv, dstv, buf, agg, sem):
        cid = lax.axis_index("c")
        sid = lax.axis_index("s")
        wid = sid * NC + cid
        r0 = sid * rows_per_tile

        pltpu.sync_copy(zero_hbm.at[pl.ds(r0, rows_per_tile)],
                        agg.at[pl.ds(r0, rows_per_tile)])
        pltpu.sync_copy(src_hbm.at[wid], srcv)
        pltpu.sync_copy(dst_hbm.at[wid], dstv)
        plsc.subcore_barrier()

        def body(c, carry):
            pltpu.async_copy(z_hbm.at[srcv.at[c]], buf, sem).wait()
            pltpu.sync_copy(buf, agg.at[dstv.at[c]], add=True)
            return carry

        lax.fori_loop(0, nchunk, body, 0)
        plsc.subcore_barrier()
        pltpu.sync_copy(agg.at[pl.ds(r0, rows_per_tile)],
                        out_hbm.at[cid, pl.ds(r0, rows_per_tile)])

    return scatter_kernel


def _stage_a(xp, degs, W1, n_pad, bm=256):
    d_in, h1 = W1.shape

    def body(x_ref, deg_ref, w_ref, z_ref, ns_ref, nd_ref):
        dsum = jnp.sum(deg_ref[...], axis=1)
        ns = lax.rsqrt(jnp.maximum(dsum[0], 1.0))
        nd = lax.rsqrt(jnp.maximum(dsum[1], 1.0))
        x = x_ref[...] * ns[:, None]
        z_ref[...] = jnp.dot(x, w_ref[...], preferred_element_type=jnp.float32)
        ns_ref[...] = ns
        nd_ref[...] = nd

    return pl.pallas_call(
        body,
        grid=(n_pad // bm,),
        in_specs=[
            pl.BlockSpec((bm, d_in), lambda i: (i, 0)),
            pl.BlockSpec((2, NW, bm), lambda i: (0, 0, i)),
            pl.BlockSpec((d_in, h1), lambda i: (0, 0)),
        ],
        out_specs=[
            pl.BlockSpec((bm, h1), lambda i: (i, 0)),
            pl.BlockSpec((bm,), lambda i: (i,)),
            pl.BlockSpec((bm,), lambda i: (i,)),
        ],
        out_shape=[
            jax.ShapeDtypeStruct((n_pad, h1), jnp.float32),
            jax.ShapeDtypeStruct((n_pad,), jnp.float32),
            jax.ShapeDtypeStruct((n_pad,), jnp.float32),
        ],
    )(xp, degs, W1)


def _stage_b(s1, ns, nd, b1, W2, n_pad, bm=256):
    h1, h2 = W2.shape

    def body(s_ref, ns_ref, nd_ref, b_ref, w_ref, z_ref):
        s = s_ref[0] + s_ref[1]
        hid = jnp.maximum(s * nd_ref[...][:, None] + b_ref[...][None, :], 0.0)
        hid = hid * ns_ref[...][:, None]
        z_ref[...] = jnp.dot(hid, w_ref[...], preferred_element_type=jnp.float32)

    return pl.pallas_call(
        body,
        grid=(n_pad // bm,),
        in_specs=[
            pl.BlockSpec((2, bm, h1), lambda i: (0, i, 0)),
            pl.BlockSpec((bm,), lambda i: (i,)),
            pl.BlockSpec((bm,), lambda i: (i,)),
            pl.BlockSpec((h1,), lambda i: (0,)),
            pl.BlockSpec((h1, h2), lambda i: (0, 0)),
        ],
        out_specs=pl.BlockSpec((bm, h2), lambda i: (i, 0)),
        out_shape=jax.ShapeDtypeStruct((n_pad, h2), jnp.float32),
    )(s1, ns, nd, b1, W2)


def _decoder(s2, nd, b2, n, bm=1024, bn=1024):
    h2 = s2.shape[-1]

    def body(sa_ref, sb_ref, na_ref, nb_ref, b_ref, o_ref):
        bias = b_ref[...][None, :]
        a = jnp.maximum((sa_ref[0] + sa_ref[1]) * na_ref[...][:, None] + bias, 0.0)
        b = jnp.maximum((sb_ref[0] + sb_ref[1]) * nb_ref[...][:, None] + bias, 0.0)
        logits = lax.dot_general(
            a, b, (((1,), (1,)), ((), ())),
            preferred_element_type=jnp.float32)
        o_ref[...] = jax.nn.sigmoid(logits)

    return pl.pallas_call(
        body,
        grid=(-(-n // bm), -(-n // bn)),
        in_specs=[
            pl.BlockSpec((2, bm, h2), lambda i, j: (0, i, 0)),
            pl.BlockSpec((2, bn, h2), lambda i, j: (0, j, 0)),
            pl.BlockSpec((bm,), lambda i, j: (i,)),
            pl.BlockSpec((bn,), lambda i, j: (j,)),
            pl.BlockSpec((h2,), lambda i, j: (0,)),
        ],
        out_specs=pl.BlockSpec((bm, bn), lambda i, j: (i, j)),
        out_shape=jax.ShapeDtypeStruct((n, n), jnp.float32),
    )(s2, s2, nd, nd, b2)


def kernel(features, edge_index, W1, b1, W2, b2):
    n, d_in = features.shape
    h1 = W1.shape[1]
    h2 = W2.shape[1]
    e = edge_index.shape[1]

    n_pad = -(-(n + 1) // 256) * 256
    nchunk = -(-e // (NW * CH))
    e_pad = NW * nchunk * CH

    src = edge_index[0].astype(jnp.int32)
    dst = edge_index[1].astype(jnp.int32)
    pad_idx = jnp.full((e_pad - e,), n, jnp.int32)
    srcp = jnp.concatenate([src, pad_idx]).reshape(NW, nchunk, CH)
    dstp = jnp.concatenate([dst, pad_idx]).reshape(NW, nchunk, CH)

    degs = _make_degrees(n_pad, nchunk)(srcp, dstp)

    z1, ns, nd = _stage_a(features, degs, W1, n_pad)

    zeros1 = jnp.zeros((n_pad, h1), jnp.float32)
    s1 = _make_scatter(n_pad, nchunk, h1)(z1, srcp, dstp, zeros1)

    z2 = _stage_b(s1, ns, nd, b1, W2, n_pad)

    zeros2 = jnp.zeros((n_pad, h2), jnp.float32)
    s2 = _make_scatter(n_pad, nchunk, h2)(z2, srcp, dstp, zeros2)

    return _decoder(s2, nd, b2, n)
